# Initial kernel scaffold; baseline (speedup 1.0000x reference)
import jax
import jax.numpy as jnp
from jax.experimental import pallas as pl


def kernel(h, edge_index, positions, dist, W, b, gamma, beta):
    raise NotImplementedError("write your pallas kernel here")


# speedup vs baseline: 10.0165x; 10.0165x over previous
import functools

import jax
import jax.numpy as jnp
from jax import lax
from jax.experimental import pallas as pl
from jax.experimental.pallas import tpu as pltpu
from jax.experimental.pallas import tpu_sc as plsc

_NC = 2
_NS = 16
_L = 16


def _matmul_call(h, wbig, bias):
    n, d_in = h.shape
    d_out = wbig.shape[1]
    d = d_out // 4
    bn = 1000

    def body(h_ref, w_ref, b_ref, base_ref, z_ref):
        y = jnp.dot(h_ref[...], w_ref[...],
                    preferred_element_type=jnp.float32) + b_ref[...]
        base_ref[...] = y[:, :d]
        z_ref[...] = y[:, d:]

    return pl.pallas_call(
        body,
        grid=(n // bn,),
        in_specs=[
            pl.BlockSpec((bn, d_in), lambda i: (i, 0)),
            pl.BlockSpec((d_in, d_out), lambda i: (0, 0)),
            pl.BlockSpec((1, d_out), lambda i: (0, 0)),
        ],
        out_specs=[
            pl.BlockSpec((bn, d), lambda i: (i, 0)),
            pl.BlockSpec((bn, 3 * d), lambda i: (i, 0)),
        ],
        out_shape=[
            jax.ShapeDtypeStruct((n, d), jnp.float32),
            jax.ShapeDtypeStruct((n, 3 * d), jnp.float32),
        ],
    )(h, wbig, bias)


def _sc_scatter_call(z3, src, dst, pos, distf, n_nodes):
    e = src.shape[0]
    d = z3.shape[1]
    nw = _NC * _NS
    epw = e // nw
    c = 80
    n_chunks = epw // c
    rps = (n_nodes // _NS) & ~7
    oc = 16
    mesh = plsc.VectorSubcoreMesh(core_axis_name="c", subcore_axis_name="s")

    @functools.partial(
        pl.kernel,
        mesh=mesh,
        out_type=jax.ShapeDtypeStruct((_NC * n_nodes, d), jnp.float32),
        scratch_types=[
            pltpu.VMEM((epw,), jnp.int32),
            pltpu.VMEM((epw,), jnp.int32),
            pltpu.VMEM((epw,), jnp.int32),
            pltpu.VMEM((epw,), jnp.float32),
            pltpu.VMEM((c,), jnp.int32),
            pltpu.VMEM((c,), jnp.int32),
            pltpu.VMEM((c,), jnp.float32),
            pltpu.VMEM((c, d), jnp.float32),
            pltpu.VMEM_SHARED((n_nodes, d), jnp.float32),
            pltpu.SemaphoreType.DMA,
        ],
    )
    def k(z_hbm, src_hbm, dst_hbm, pos_hbm, dist_hbm, out_hbm,
          src_a, dst_a, pos_a, dist_a, idx_v, sct_v, scl_v, rows_v,
          acc_sh, sem):
        cid = lax.axis_index("c")
        sid = lax.axis_index("s")
        wid = cid * _NS + sid
        base = wid * epw

        pltpu.sync_copy(src_hbm.at[pl.ds(base, epw)], src_a)
        pltpu.sync_copy(dst_hbm.at[pl.ds(base, epw)], dst_a)
        pltpu.sync_copy(pos_hbm.at[pl.ds(base, epw)], pos_a)
        pltpu.sync_copy(dist_hbm.at[pl.ds(base, epw)], dist_a)

        def zero_rows(j, carry):
            for kk in range(d // _L):
                rows_v[j, pl.ds(kk * _L, _L)] = jnp.zeros((_L,), jnp.float32)
            return carry

        lax.fori_loop(0, oc, zero_rows, 0)
        r0 = sid * rps
        nrows = jnp.where(sid == _NS - 1, n_nodes - (_NS - 1) * rps, rps)

        def zcopy(i, carry):
            pltpu.sync_copy(rows_v.at[pl.ds(0, oc)],
                            acc_sh.at[pl.ds(r0 + i * oc, oc)])
            return carry

        lax.fori_loop(0, nrows // oc, zcopy, 0)
        plsc.subcore_barrier()

        def chunk_body(t, carry):
            off = t * c
            for i in range(c // _L):
                sl = pl.ds(off + i * _L, _L)
                s16 = src_a[sl]
                p16 = pos_a[sl]
                d16 = dist_a[sl]
                idx_v[pl.ds(i * _L, _L)] = s16 * 3 + jnp.minimum(p16, 2)
                scl_v[pl.ds(i * _L, _L)] = jnp.where(p16 == 3, 0.0, d16)
                sct_v[pl.ds(i * _L, _L)] = dst_a[sl]
            pltpu.async_copy(z_hbm.at[idx_v], rows_v, sem).wait()

            def scale_grp(g, cry):
                s16 = scl_v[pl.ds(g * _L, _L)]
                for jj in range(_L):
                    s = s16[jj]
                    j = g * _L + jj
                    for kk in range(d // _L):
                        rows_v[j, pl.ds(kk * _L, _L)] = (
                            rows_v[j, pl.ds(kk * _L, _L)] * s)
                return cry

            lax.fori_loop(0, c // _L, scale_grp, 0)
            pltpu.sync_copy(rows_v, acc_sh.at[sct_v], add=True)
            return carry

        lax.fori_loop(0, n_chunks, chunk_body, 0)
        plsc.subcore_barrier()

        def out_copy(i, carry):
            pltpu.sync_copy(acc_sh.at[pl.ds(r0 + i * oc, oc)],
                            rows_v.at[pl.ds(0, oc)])
            pltpu.sync_copy(
                rows_v.at[pl.ds(0, oc)],
                out_hbm.at[pl.ds(cid * n_nodes + r0 + i * oc, oc)])
            return carry

        lax.fori_loop(0, nrows // oc, out_copy, 0)

    return k(z3, src, dst, pos, distf)


def _epilogue_call(basearr, partials, gamma, beta):
    n, d = basearr.shape
    bn = 1000
    nblk = n // bn

    def body(b_ref, p0_ref, p1_ref, g_ref, be_ref, o_ref):
        y = b_ref[...] + p0_ref[...] + p1_ref[...]
        mu = jnp.mean(y, axis=-1, keepdims=True)
        var = jnp.mean(jnp.square(y - mu), axis=-1, keepdims=True)
        yn = (y - mu) * lax.rsqrt(var + 1e-5) * g_ref[...] + be_ref[...]
        o_ref[...] = jnp.maximum(yn, 0.0)

    return pl.pallas_call(
        body,
        grid=(nblk,),
        in_specs=[
            pl.BlockSpec((bn, d), lambda i: (i, 0)),
            pl.BlockSpec((bn, d), lambda i: (i, 0)),
            pl.BlockSpec((bn, d), lambda i: (i + nblk, 0)),
            pl.BlockSpec((1, d), lambda i: (0, 0)),
            pl.BlockSpec((1, d), lambda i: (0, 0)),
        ],
        out_specs=pl.BlockSpec((bn, d), lambda i: (i, 0)),
        out_shape=jax.ShapeDtypeStruct((n, d), jnp.float32),
    )(basearr, partials, partials, gamma.reshape(1, d), beta.reshape(1, d))


def kernel(h, edge_index, positions, dist, W, b, gamma, beta):
    n, d = h.shape
    wstack = jnp.concatenate(
        [W[:, :d], W[:, d:2 * d] + W[:, 4 * d:], W[:, 2 * d:3 * d],
         W[:, 3 * d:4 * d]], axis=0)
    wbig = wstack.T
    bias = jnp.concatenate(
        [b, jnp.zeros((3 * d,), jnp.float32)]).reshape(1, 4 * d)

    basearr, z = _matmul_call(h, wbig, bias)
    z3 = z.reshape(3 * n, d)

    src = edge_index[0]
    dst = edge_index[1]
    distf = dist.reshape(-1)
    partials = _sc_scatter_call(z3, src, dst, positions, distf, n)
    return _epilogue_call(basearr, partials, gamma, beta)
